# Initial kernel scaffold; baseline (speedup 1.0000x reference)
import jax
import jax.numpy as jnp
from jax.experimental import pallas as pl


def kernel(x_nchw, w0, b0, w1, b1):
    raise NotImplementedError("write your pallas kernel here")


# speedup vs baseline: 1.4363x; 1.4363x over previous
import jax
import jax.numpy as jnp
from jax.experimental import pallas as pl
from jax.experimental.pallas import tpu as pltpu


def _fused_vgg_kernel(x_ref, w0_ref, w1p_ref, w1s_ref, b0_ref, b1_ref,
                      o_ref, x3_ref, c1_ref, x2_ref, acc_ref):
    Hp = x_ref.shape[1]
    Wp = x_ref.shape[2]
    H, W = Hp - 2, Wp - 2
    Cin = x_ref.shape[3]
    C = o_ref.shape[-1]

    x3_ref[...] = jnp.concatenate(
        [x_ref[0, :, dx:dx + W, :] for dx in range(3)], axis=-1)
    for dy in range(3):
        d = jnp.dot(x3_ref[dy:dy + H].reshape(H * W, 3 * Cin), w0_ref[dy],
                    preferred_element_type=jnp.float32)
        if dy == 0:
            acc_ref[...] = d
        else:
            acc_ref[...] += d

    a1 = jnp.maximum(acc_ref[...] + b0_ref[...], 0.0).astype(jnp.bfloat16)
    c1_ref[0:1, :, :] = jnp.zeros((1, Wp, C), jnp.bfloat16)
    c1_ref[Hp - 1:Hp, :, :] = jnp.zeros((1, Wp, C), jnp.bfloat16)
    c1_ref[:, 0:1, :] = jnp.zeros((Hp, 1, C), jnp.bfloat16)
    c1_ref[:, Wp - 1:Wp, :] = jnp.zeros((Hp, 1, C), jnp.bfloat16)
    c1_ref[1:H + 1, 1:W + 1, :] = a1.reshape(H, W, C)

    x2_ref[...] = jnp.concatenate(
        [c1_ref[:, 0:W, :], c1_ref[:, 1:W + 1, :]], axis=-1)
    for dy in range(3):
        d = jnp.dot(x2_ref[dy:dy + H].reshape(H * W, 2 * C), w1p_ref[dy],
                    preferred_element_type=jnp.float32)
        e = jnp.dot(c1_ref[dy:dy + H, 2:W + 2, :].reshape(H * W, C),
                    w1s_ref[dy], preferred_element_type=jnp.float32)
        if dy == 0:
            acc_ref[...] = d + e
        else:
            acc_ref[...] += d + e

    a2 = jnp.maximum(acc_ref[...] + b1_ref[...], 0.0)
    r2 = a2.reshape(H // 2, 2, W // 2, 2 * C)
    hp = jnp.maximum(r2[:, 0], r2[:, 1])
    o_ref[0] = jnp.maximum(hp[..., :C], hp[..., C:]).astype(o_ref.dtype)


def kernel(x_nchw, w0, b0, w1, b1):
    N, Cin, H, W = x_nchw.shape
    C = w1.shape[-1]
    Hp, Wp = H + 2, W + 2

    x = jnp.transpose(x_nchw, (0, 2, 3, 1)).astype(jnp.bfloat16)
    xh = jnp.pad(x, ((0, 0), (1, 1), (1, 1), (0, 0)))

    w0r = w0.astype(jnp.bfloat16).reshape(3, 3 * Cin, C)
    w1p = w1[:, 0:2].astype(jnp.bfloat16).reshape(3, 2 * C, C)
    w1s = w1[:, 2].astype(jnp.bfloat16)
    b0r = b0.reshape(1, C).astype(jnp.float32)
    b1r = b1.reshape(1, C).astype(jnp.float32)

    y = pl.pallas_call(
        _fused_vgg_kernel,
        out_shape=jax.ShapeDtypeStruct((N, H // 2, W // 2, C), jnp.bfloat16),
        grid=(N,),
        in_specs=[
            pl.BlockSpec((1, Hp, Wp, Cin), lambda n: (n, 0, 0, 0)),
            pl.BlockSpec((3, 3 * Cin, C), lambda n: (0, 0, 0)),
            pl.BlockSpec((3, 2 * C, C), lambda n: (0, 0, 0)),
            pl.BlockSpec((3, C, C), lambda n: (0, 0, 0)),
            pl.BlockSpec((1, C), lambda n: (0, 0)),
            pl.BlockSpec((1, C), lambda n: (0, 0)),
        ],
        out_specs=pl.BlockSpec((1, H // 2, W // 2, C), lambda n: (n, 0, 0, 0)),
        scratch_shapes=[
            pltpu.VMEM((Hp, W, 3 * Cin), jnp.bfloat16),
            pltpu.VMEM((Hp, Wp, C), jnp.bfloat16),
            pltpu.VMEM((Hp, W, 2 * C), jnp.bfloat16),
            pltpu.VMEM((H * W, C), jnp.float32),
        ],
        compiler_params=pltpu.CompilerParams(
            dimension_semantics=("parallel",)),
    )(xh, w0r, w1p, w1s, b0r, b1r)

    return jnp.transpose(y, (0, 3, 1, 2))


# speedup vs baseline: 1.8452x; 1.2847x over previous
import jax
import jax.numpy as jnp
from jax.experimental import pallas as pl
from jax.experimental.pallas import tpu as pltpu


def _fused_vgg_kernel(x_ref, w0p_ref, w0s_ref, w1p_ref, w1q_ref,
                      w1sp_ref, w1ss_ref, b0_ref, b1_ref,
                      o_ref, x3_ref, c1_ref, x2_ref, p_ref, acc_ref):
    Hp = x_ref.shape[1]
    Wp = x_ref.shape[2]
    H, W = Hp - 2, Wp - 2
    Cin = x_ref.shape[3]
    C = o_ref.shape[-1]
    M2 = Hp * W
    M1 = H * W

    x3_ref[...] = jnp.concatenate(
        [x_ref[0, :, dx:dx + W, :] for dx in range(3)], axis=-1)
    p_ref[...] = jnp.dot(
        x3_ref[...].reshape(M2, 3 * Cin), w0p_ref[...],
        preferred_element_type=jnp.float32).reshape(Hp, W, 2 * C)
    acc_ref[...] = (p_ref[0:H, :, :C] + p_ref[1:H + 1, :, C:]
                    ).reshape(M1, C)
    acc_ref[...] += jnp.dot(
        x3_ref[2:H + 2].reshape(M1, 3 * Cin), w0s_ref[...],
        preferred_element_type=jnp.float32)

    a1 = jnp.maximum(acc_ref[...] + b0_ref[...], 0.0).astype(jnp.bfloat16)
    c1_ref[0:1, :, :] = jnp.zeros((1, Wp, C), jnp.bfloat16)
    c1_ref[Hp - 1:Hp, :, :] = jnp.zeros((1, Wp, C), jnp.bfloat16)
    c1_ref[:, 0:1, :] = jnp.zeros((Hp, 1, C), jnp.bfloat16)
    c1_ref[:, Wp - 1:Wp, :] = jnp.zeros((Hp, 1, C), jnp.bfloat16)
    c1_ref[1:H + 1, 1:W + 1, :] = a1.reshape(H, W, C)

    x2_ref[...] = jnp.concatenate(
        [c1_ref[:, 0:W, :], c1_ref[:, 1:W + 1, :]], axis=-1)
    p_ref[...] = jnp.dot(
        x2_ref[...].reshape(M2, 2 * C), w1p_ref[...],
        preferred_element_type=jnp.float32).reshape(Hp, W, 2 * C)
    acc_ref[...] = (p_ref[0:H, :, :C] + p_ref[1:H + 1, :, C:]
                    ).reshape(M1, C)
    acc_ref[...] += jnp.dot(
        x2_ref[2:H + 2].reshape(M1, 2 * C), w1q_ref[...],
        preferred_element_type=jnp.float32)
    p_ref[...] = jnp.dot(
        c1_ref[:, 2:W + 2, :].reshape(M2, C), w1sp_ref[...],
        preferred_element_type=jnp.float32).reshape(Hp, W, 2 * C)
    acc_ref[...] += (p_ref[0:H, :, :C] + p_ref[1:H + 1, :, C:]
                     ).reshape(M1, C)
    acc_ref[...] += jnp.dot(
        c1_ref[2:H + 2, 2:W + 2, :].reshape(M1, C), w1ss_ref[...],
        preferred_element_type=jnp.float32)

    a2 = jnp.maximum(acc_ref[...] + b1_ref[...], 0.0)
    r2 = a2.reshape(H // 2, 2, W // 2, 2 * C)
    hp = jnp.maximum(r2[:, 0], r2[:, 1])
    o_ref[0] = jnp.maximum(hp[..., :C], hp[..., C:]).astype(o_ref.dtype)


def kernel(x_nchw, w0, b0, w1, b1):
    N, Cin, H, W = x_nchw.shape
    C = w1.shape[-1]
    Hp, Wp = H + 2, W + 2

    x = jnp.transpose(x_nchw, (0, 2, 3, 1)).astype(jnp.bfloat16)
    xh = jnp.pad(x, ((0, 0), (1, 1), (1, 1), (0, 0)))

    w0r = w0.astype(jnp.bfloat16).reshape(3, 3 * Cin, C)
    w0p = jnp.concatenate([w0r[0], w0r[1]], axis=-1)
    w0s = w0r[2]
    w1r = w1[:, 0:2].astype(jnp.bfloat16).reshape(3, 2 * C, C)
    w1p = jnp.concatenate([w1r[0], w1r[1]], axis=-1)
    w1q = w1r[2]
    w1s = w1[:, 2].astype(jnp.bfloat16)
    w1sp = jnp.concatenate([w1s[0], w1s[1]], axis=-1)
    w1ss = w1s[2]
    b0r = b0.reshape(1, C).astype(jnp.float32)
    b1r = b1.reshape(1, C).astype(jnp.float32)

    y = pl.pallas_call(
        _fused_vgg_kernel,
        out_shape=jax.ShapeDtypeStruct((N, H // 2, W // 2, C), jnp.bfloat16),
        grid=(N,),
        in_specs=[
            pl.BlockSpec((1, Hp, Wp, Cin), lambda n: (n, 0, 0, 0)),
            pl.BlockSpec((3 * Cin, 2 * C), lambda n: (0, 0)),
            pl.BlockSpec((3 * Cin, C), lambda n: (0, 0)),
            pl.BlockSpec((2 * C, 2 * C), lambda n: (0, 0)),
            pl.BlockSpec((2 * C, C), lambda n: (0, 0)),
            pl.BlockSpec((C, 2 * C), lambda n: (0, 0)),
            pl.BlockSpec((C, C), lambda n: (0, 0)),
            pl.BlockSpec((1, C), lambda n: (0, 0)),
            pl.BlockSpec((1, C), lambda n: (0, 0)),
        ],
        out_specs=pl.BlockSpec((1, H // 2, W // 2, C), lambda n: (n, 0, 0, 0)),
        scratch_shapes=[
            pltpu.VMEM((Hp, W, 3 * Cin), jnp.bfloat16),
            pltpu.VMEM((Hp, Wp, C), jnp.bfloat16),
            pltpu.VMEM((Hp, W, 2 * C), jnp.bfloat16),
            pltpu.VMEM((Hp, W, 2 * C), jnp.float32),
            pltpu.VMEM((H * W, C), jnp.float32),
        ],
        compiler_params=pltpu.CompilerParams(
            dimension_semantics=("parallel",)),
    )(xh, w0p, w0s, w1p, w1q, w1sp, w1ss, b0r, b1r)

    return jnp.transpose(y, (0, 3, 1, 2))


# speedup vs baseline: 2.9049x; 1.5743x over previous
import jax
import jax.numpy as jnp
from jax.experimental import pallas as pl
from jax.experimental.pallas import tpu as pltpu


def _fused_vgg_kernel(x_ref, w0p_ref, w0s_ref, w1p_ref, w1q_ref,
                      w1sp_ref, w1ss_ref, b0_ref, b1_ref,
                      o_ref, xh_ref, x3_ref, c1_ref, x2_ref, p_ref, acc_ref):
    Cin = x_ref.shape[1]
    H = x_ref.shape[2]
    W = x_ref.shape[3]
    Hp, Wp = H + 2, W + 2
    C = o_ref.shape[-1]
    M2 = Hp * W
    M1 = H * W

    xh_ref[0:1, :, :] = jnp.zeros((1, Wp, Cin), jnp.bfloat16)
    xh_ref[Hp - 1:Hp, :, :] = jnp.zeros((1, Wp, Cin), jnp.bfloat16)
    xh_ref[:, 0:1, :] = jnp.zeros((Hp, 1, Cin), jnp.bfloat16)
    xh_ref[:, Wp - 1:Wp, :] = jnp.zeros((Hp, 1, Cin), jnp.bfloat16)
    xh_ref[1:H + 1, 1:W + 1, :] = jnp.transpose(
        x_ref[0].astype(jnp.bfloat16), (1, 2, 0))

    x3_ref[...] = jnp.concatenate(
        [xh_ref[:, dx:dx + W, :] for dx in range(3)], axis=-1)
    p_ref[...] = jnp.dot(
        x3_ref[...].reshape(M2, 3 * Cin), w0p_ref[...],
        preferred_element_type=jnp.float32).reshape(Hp, W, 2 * C)
    acc_ref[...] = (p_ref[0:H, :, :C] + p_ref[1:H + 1, :, C:]
                    ).reshape(M1, C)
    acc_ref[...] += jnp.dot(
        x3_ref[2:H + 2].reshape(M1, 3 * Cin), w0s_ref[...],
        preferred_element_type=jnp.float32)

    a1 = jnp.maximum(acc_ref[...] + b0_ref[...], 0.0).astype(jnp.bfloat16)
    c1_ref[0:1, :, :] = jnp.zeros((1, Wp, C), jnp.bfloat16)
    c1_ref[Hp - 1:Hp, :, :] = jnp.zeros((1, Wp, C), jnp.bfloat16)
    c1_ref[:, 0:1, :] = jnp.zeros((Hp, 1, C), jnp.bfloat16)
    c1_ref[:, Wp - 1:Wp, :] = jnp.zeros((Hp, 1, C), jnp.bfloat16)
    c1_ref[1:H + 1, 1:W + 1, :] = a1.reshape(H, W, C)

    x2_ref[...] = jnp.concatenate(
        [c1_ref[:, 0:W, :], c1_ref[:, 1:W + 1, :]], axis=-1)
    p_ref[...] = jnp.dot(
        x2_ref[...].reshape(M2, 2 * C), w1p_ref[...],
        preferred_element_type=jnp.float32).reshape(Hp, W, 2 * C)
    acc_ref[...] = (p_ref[0:H, :, :C] + p_ref[1:H + 1, :, C:]
                    ).reshape(M1, C)
    acc_ref[...] += jnp.dot(
        x2_ref[2:H + 2].reshape(M1, 2 * C), w1q_ref[...],
        preferred_element_type=jnp.float32)
    p_ref[...] = jnp.dot(
        c1_ref[:, 2:W + 2, :].reshape(M2, C), w1sp_ref[...],
        preferred_element_type=jnp.float32).reshape(Hp, W, 2 * C)
    acc_ref[...] += (p_ref[0:H, :, :C] + p_ref[1:H + 1, :, C:]
                     ).reshape(M1, C)
    acc_ref[...] += jnp.dot(
        c1_ref[2:H + 2, 2:W + 2, :].reshape(M1, C), w1ss_ref[...],
        preferred_element_type=jnp.float32)

    a2 = jnp.maximum(acc_ref[...] + b1_ref[...], 0.0)
    r2 = a2.reshape(H // 2, 2, W // 2, 2 * C)
    hp = jnp.maximum(r2[:, 0], r2[:, 1])
    o_ref[0] = jnp.maximum(hp[..., :C], hp[..., C:]).astype(o_ref.dtype)


def kernel(x_nchw, w0, b0, w1, b1):
    N, Cin, H, W = x_nchw.shape
    C = w1.shape[-1]
    Hp, Wp = H + 2, W + 2

    w0r = w0.astype(jnp.bfloat16).reshape(3, 3 * Cin, C)
    w0p = jnp.concatenate([w0r[0], w0r[1]], axis=-1)
    w0s = w0r[2]
    w1r = w1[:, 0:2].astype(jnp.bfloat16).reshape(3, 2 * C, C)
    w1p = jnp.concatenate([w1r[0], w1r[1]], axis=-1)
    w1q = w1r[2]
    w1s = w1[:, 2].astype(jnp.bfloat16)
    w1sp = jnp.concatenate([w1s[0], w1s[1]], axis=-1)
    w1ss = w1s[2]
    b0r = b0.reshape(1, C).astype(jnp.float32)
    b1r = b1.reshape(1, C).astype(jnp.float32)

    y = pl.pallas_call(
        _fused_vgg_kernel,
        out_shape=jax.ShapeDtypeStruct((N, H // 2, W // 2, C), jnp.bfloat16),
        grid=(N,),
        in_specs=[
            pl.BlockSpec((1, Cin, H, W), lambda n: (n, 0, 0, 0)),
            pl.BlockSpec((3 * Cin, 2 * C), lambda n: (0, 0)),
            pl.BlockSpec((3 * Cin, C), lambda n: (0, 0)),
            pl.BlockSpec((2 * C, 2 * C), lambda n: (0, 0)),
            pl.BlockSpec((2 * C, C), lambda n: (0, 0)),
            pl.BlockSpec((C, 2 * C), lambda n: (0, 0)),
            pl.BlockSpec((C, C), lambda n: (0, 0)),
            pl.BlockSpec((1, C), lambda n: (0, 0)),
            pl.BlockSpec((1, C), lambda n: (0, 0)),
        ],
        out_specs=pl.BlockSpec((1, H // 2, W // 2, C), lambda n: (n, 0, 0, 0)),
        scratch_shapes=[
            pltpu.VMEM((Hp, Wp, Cin), jnp.bfloat16),
            pltpu.VMEM((Hp, W, 3 * Cin), jnp.bfloat16),
            pltpu.VMEM((Hp, Wp, C), jnp.bfloat16),
            pltpu.VMEM((Hp, W, 2 * C), jnp.bfloat16),
            pltpu.VMEM((Hp, W, 2 * C), jnp.float32),
            pltpu.VMEM((H * W, C), jnp.float32),
        ],
        compiler_params=pltpu.CompilerParams(
            dimension_semantics=("parallel",)),
    )(x_nchw, w0p, w0s, w1p, w1q, w1sp, w1ss, b0r, b1r)

    return jnp.transpose(y, (0, 3, 1, 2))


# speedup vs baseline: 2.9072x; 1.0008x over previous
import jax
import jax.numpy as jnp
from jax.experimental import pallas as pl
from jax.experimental.pallas import tpu as pltpu


def _fused_vgg_kernel(x_ref, w0_ref, w1a_ref, w1b_ref, b0_ref, b1_ref,
                      o_ref, xh_ref, cat_ref, c1_ref, p_ref, acc_ref):
    Cin = x_ref.shape[1]
    H = x_ref.shape[2]
    W = x_ref.shape[3]
    Hp = H + 2
    C = o_ref.shape[-1]
    M2 = Hp * W
    M1 = H * W
    L = 8
    Z = jnp.zeros((Hp, 1, Cin), jnp.bfloat16)

    xh_ref[0:1, 7:9 + W, :] = jnp.zeros((1, W + 2, Cin), jnp.bfloat16)
    xh_ref[Hp - 1:Hp, 7:9 + W, :] = jnp.zeros((1, W + 2, Cin), jnp.bfloat16)
    xh_ref[:, 7:8, :] = Z
    xh_ref[:, 8 + W:9 + W, :] = Z
    t1 = jnp.transpose(x_ref[0].astype(jnp.bfloat16), (1, 0, 2))
    xh_ref[1:H + 1, L:L + W, :] = jnp.transpose(t1, (0, 2, 1))

    cat_ref[:, :, 0:3 * Cin] = jnp.concatenate(
        [xh_ref[:, 7 + dx:7 + dx + W, :] for dx in range(3)], axis=-1)
    cat_ref[:, :, 3 * Cin:] = jnp.zeros((Hp, W, 2 * C - 3 * Cin),
                                        jnp.bfloat16)
    p_ref[...] = jnp.dot(
        cat_ref[...].reshape(M2, 2 * C), w0_ref[...],
        preferred_element_type=jnp.float32).reshape(Hp, W, 3 * C)
    acc_ref[...] = (p_ref[0:H, :, 0:C] + p_ref[1:H + 1, :, C:2 * C]
                    + p_ref[2:H + 2, :, 2 * C:]).reshape(M1, C)

    a1 = jnp.maximum(acc_ref[...] + b0_ref[...], 0.0).astype(jnp.bfloat16)
    c1_ref[0:1, 7:9 + W, :] = jnp.zeros((1, W + 2, C), jnp.bfloat16)
    c1_ref[Hp - 1:Hp, 7:9 + W, :] = jnp.zeros((1, W + 2, C), jnp.bfloat16)
    c1_ref[:, 7:8, :] = jnp.zeros((Hp, 1, C), jnp.bfloat16)
    c1_ref[:, 8 + W:9 + W, :] = jnp.zeros((Hp, 1, C), jnp.bfloat16)
    c1_ref[1:H + 1, L:L + W, :] = a1.reshape(H, W, C)

    cat_ref[...] = jnp.concatenate(
        [c1_ref[:, 7:7 + W, :], c1_ref[:, 8:8 + W, :]], axis=-1)
    p_ref[...] = jnp.dot(
        cat_ref[...].reshape(M2, 2 * C), w1a_ref[...],
        preferred_element_type=jnp.float32).reshape(Hp, W, 3 * C)
    acc_ref[...] = (p_ref[0:H, :, 0:C] + p_ref[1:H + 1, :, C:2 * C]
                    + p_ref[2:H + 2, :, 2 * C:]).reshape(M1, C)
    p_ref[...] = jnp.dot(
        c1_ref[:, 9:9 + W, :].reshape(M2, C), w1b_ref[...],
        preferred_element_type=jnp.float32).reshape(Hp, W, 3 * C)
    acc_ref[...] += (p_ref[0:H, :, 0:C] + p_ref[1:H + 1, :, C:2 * C]
                     + p_ref[2:H + 2, :, 2 * C:]).reshape(M1, C)

    a2 = jnp.maximum(acc_ref[...] + b1_ref[...], 0.0)
    r2 = a2.reshape(H // 2, 2, W // 2, 2 * C)
    hp = jnp.maximum(r2[:, 0], r2[:, 1])
    o_ref[0] = jnp.maximum(hp[..., :C], hp[..., C:]).astype(o_ref.dtype)


def kernel(x_nchw, w0, b0, w1, b1):
    N, Cin, H, W = x_nchw.shape
    C = w1.shape[-1]
    Hp = H + 2

    w0r = w0.astype(jnp.bfloat16).reshape(3, 3 * Cin, C)
    w0_all = jnp.concatenate([w0r[0], w0r[1], w0r[2]], axis=-1)
    w0_all = jnp.pad(w0_all, ((0, 2 * C - 3 * Cin), (0, 0)))
    w1r = w1[:, 0:2].astype(jnp.bfloat16).reshape(3, 2 * C, C)
    w1a = jnp.concatenate([w1r[0], w1r[1], w1r[2]], axis=-1)
    w1s = w1[:, 2].astype(jnp.bfloat16)
    w1b = jnp.concatenate([w1s[0], w1s[1], w1s[2]], axis=-1)
    b0r = b0.reshape(1, C).astype(jnp.float32)
    b1r = b1.reshape(1, C).astype(jnp.float32)

    y = pl.pallas_call(
        _fused_vgg_kernel,
        out_shape=jax.ShapeDtypeStruct((N, H // 2, W // 2, C), jnp.bfloat16),
        grid=(N,),
        in_specs=[
            pl.BlockSpec((1, Cin, H, W), lambda n: (n, 0, 0, 0)),
            pl.BlockSpec((2 * C, 3 * C), lambda n: (0, 0)),
            pl.BlockSpec((2 * C, 3 * C), lambda n: (0, 0)),
            pl.BlockSpec((C, 3 * C), lambda n: (0, 0)),
            pl.BlockSpec((1, C), lambda n: (0, 0)),
            pl.BlockSpec((1, C), lambda n: (0, 0)),
        ],
        out_specs=pl.BlockSpec((1, H // 2, W // 2, C), lambda n: (n, 0, 0, 0)),
        scratch_shapes=[
            pltpu.VMEM((Hp, 128, Cin), jnp.bfloat16),
            pltpu.VMEM((Hp, W, 2 * C), jnp.bfloat16),
            pltpu.VMEM((Hp, 128, C), jnp.bfloat16),
            pltpu.VMEM((Hp, W, 3 * C), jnp.float32),
            pltpu.VMEM((H * W, C), jnp.float32),
        ],
        compiler_params=pltpu.CompilerParams(
            dimension_semantics=("parallel",)),
    )(x_nchw, w0_all, w1a, w1b, b0r, b1r)

    return jnp.transpose(y, (0, 3, 1, 2))
